# baseline (speedup 1.0000x reference)
import functools

import jax
import jax.numpy as jnp
from jax import lax
from jax.experimental import pallas as pl
from jax.experimental.pallas import tpu as pltpu
from jax.experimental.pallas import tpu_sc as plsc

_L = 16
_CH = 8192
_NACC = 8


@functools.cache
def _build(n):
  info = plsc.get_sparse_core_info()
  nc = info.num_cores
  nw = nc * info.num_subcores
  n_full = n // _CH
  rem = n - n_full * _CH
  rem_vecs = (rem + _L - 1) // _L
  mesh = plsc.VectorSubcoreMesh(core_axis_name="c", subcore_axis_name="s")

  @functools.partial(
      pl.kernel,
      mesh=mesh,
      out_type=jax.ShapeDtypeStruct((nw, _L), jnp.float32),
      scratch_types=[
          pltpu.VMEM((_CH,), jnp.float32),
          pltpu.VMEM((_L,), jnp.float32),
          pltpu.SemaphoreType.DMA,
      ],
  )
  def ksum(vals, out, buf, stage, sem):
    wid = lax.axis_index("s") * nc + lax.axis_index("c")
    zero = jnp.zeros((_L,), jnp.float32)

    def chunk_body(i, accs):
      c = wid + i * nw
      pltpu.async_copy(vals.at[pl.ds(c * _CH, _CH)], buf, sem).wait()

      def vec_body(j, accs):
        base = j * (_L * _NACC)
        return tuple(
            a + buf[pl.ds(base + k * _L, _L)] for k, a in enumerate(accs))

      return lax.fori_loop(0, _CH // (_L * _NACC), vec_body, accs)

    nchunks = (n_full + nw - 1 - wid) // nw
    accs = lax.fori_loop(0, nchunks, chunk_body, (zero,) * _NACC)

    while len(accs) > 1:
      accs = tuple(accs[i] + accs[i + 1] for i in range(0, len(accs), 2))
    stage[...] = accs[0]

    if rem:
      @pl.when(wid == nw - 1)
      def _():
        buf[pl.ds(rem_vecs * _L - _L, _L)] = zero
        pltpu.async_copy(
            vals.at[pl.ds(n_full * _CH, rem)], buf.at[pl.ds(0, rem)], sem
        ).wait()

        def rbody(j, r):
          return r + buf[pl.ds(j * _L, _L)]

        r = lax.fori_loop(0, rem_vecs, rbody, zero)
        stage[...] = stage[...] + r

    pltpu.sync_copy(stage, out.at[wid])

  return ksum


def kernel(values, indices):
  del indices
  partials = _build(values.shape[0])(values)
  return jnp.sum(partials)
